# Initial kernel scaffold; baseline (speedup 1.0000x reference)
import jax
import jax.numpy as jnp
from jax.experimental import pallas as pl


def kernel(flat_features, cu_seqlens, gamma, beta, W, b):
    raise NotImplementedError("write your pallas kernel here")


# speedup vs baseline: 8.9686x; 8.9686x over previous
import functools

import jax
import jax.numpy as jnp
from jax.experimental import pallas as pl
from jax.experimental.pallas import tpu as pltpu

B = 8
EPS = 1e-5


def _onehot(cu_ref, row0, rows, nseg):
    r = jax.lax.broadcasted_iota(jnp.int32, (rows, 1), 0) + row0
    cols = []
    for s in range(nseg):
        lo = cu_ref[s]
        hi = cu_ref[s + 1]
        cols.append(((r >= lo) & (r < hi)).astype(jnp.float32))
    return jnp.concatenate(cols, axis=1)


def _stats_kernel(cu_ref, x_ref, gamma_ref, beta_ref, wt_ref, b_ref,
                  scale_ref, bias2_ref, sums_ref, sq_ref, *, rows, nblocks):
    i = pl.program_id(0)

    @pl.when(i == 0)
    def _init():
        sums_ref[...] = jnp.zeros_like(sums_ref)
        sq_ref[...] = jnp.zeros_like(sq_ref)

    x = x_ref[...]
    oh = _onehot(cu_ref, i * rows, rows, B)
    ohT = oh.T
    sums_ref[...] += jax.lax.dot(ohT, x, preferred_element_type=jnp.float32)
    sq_ref[...] += jax.lax.dot(ohT, x * x, preferred_element_type=jnp.float32)

    @pl.when(i == nblocks - 1)
    def _finalize():
        cnts = []
        for s in range(B):
            cnts.append((cu_ref[s + 1] - cu_ref[s]).astype(jnp.float32))
        counts = jnp.stack(cnts).reshape(B, 1)
        mean = sums_ref[...] / counts
        var = sq_ref[...] / counts - mean * mean
        scale = gamma_ref[...] * jax.lax.rsqrt(var + EPS)
        bias = beta_ref[...] - mean * scale
        scale_ref[...] = scale
        bias2_ref[...] = (
            jax.lax.dot(bias, wt_ref[...], preferred_element_type=jnp.float32)
            + b_ref[...])


def _apply_kernel(cu_ref, x_ref, scale_ref, bias2_ref, wt_ref, y_ref, *, rows):
    i = pl.program_id(0)
    oh = _onehot(cu_ref, i * rows, rows, B)
    scale_b = jax.lax.dot(oh, scale_ref[...], preferred_element_type=jnp.float32)
    bias_b = jax.lax.dot(oh, bias2_ref[...], preferred_element_type=jnp.float32)
    xs = x_ref[...] * scale_b
    y_ref[...] = jax.lax.dot(xs, wt_ref[...],
                             preferred_element_type=jnp.float32) + bias_b


@jax.jit
def kernel(flat_features, cu_seqlens, gamma, beta, W, b):
    T, C = flat_features.shape
    WT = W.T
    gamma2 = gamma.reshape(1, C)
    beta2 = beta.reshape(1, C)
    b2 = b.reshape(1, C)

    rows1 = 2048
    nb1 = T // rows1
    grid_spec = pltpu.PrefetchScalarGridSpec(
        num_scalar_prefetch=1,
        grid=(nb1,),
        in_specs=[
            pl.BlockSpec((rows1, C), lambda i, cu: (i, 0)),
            pl.BlockSpec((1, C), lambda i, cu: (0, 0)),
            pl.BlockSpec((1, C), lambda i, cu: (0, 0)),
            pl.BlockSpec((C, C), lambda i, cu: (0, 0)),
            pl.BlockSpec((1, C), lambda i, cu: (0, 0)),
        ],
        out_specs=[
            pl.BlockSpec((B, C), lambda i, cu: (0, 0)),
            pl.BlockSpec((B, C), lambda i, cu: (0, 0)),
        ],
        scratch_shapes=[
            pltpu.VMEM((B, C), jnp.float32),
            pltpu.VMEM((B, C), jnp.float32),
        ],
    )
    scale, bias2 = pl.pallas_call(
        functools.partial(_stats_kernel, rows=rows1, nblocks=nb1),
        grid_spec=grid_spec,
        out_shape=[
            jax.ShapeDtypeStruct((B, C), jnp.float32),
            jax.ShapeDtypeStruct((B, C), jnp.float32),
        ],
    )(cu_seqlens, flat_features, gamma2, beta2, WT, b2)

    rows2 = 1024
    nb2 = T // rows2
    grid_spec2 = pltpu.PrefetchScalarGridSpec(
        num_scalar_prefetch=1,
        grid=(nb2,),
        in_specs=[
            pl.BlockSpec((rows2, C), lambda i, cu: (i, 0)),
            pl.BlockSpec((B, C), lambda i, cu: (0, 0)),
            pl.BlockSpec((B, C), lambda i, cu: (0, 0)),
            pl.BlockSpec((C, C), lambda i, cu: (0, 0)),
        ],
        out_specs=pl.BlockSpec((rows2, C), lambda i, cu: (i, 0)),
    )
    y = pl.pallas_call(
        functools.partial(_apply_kernel, rows=rows2),
        grid_spec=grid_spec2,
        out_shape=jax.ShapeDtypeStruct((T, C), jnp.float32),
    )(cu_seqlens, flat_features, scale, bias2, WT)
    return y


# speedup vs baseline: 9.6635x; 1.0775x over previous
import functools

import jax
import jax.numpy as jnp
from jax.experimental import pallas as pl
from jax.experimental.pallas import tpu as pltpu

B = 8
EPS = 1e-5


def _onehot(cu_ref, row0, rows, nseg):
    r = jax.lax.broadcasted_iota(jnp.int32, (rows, 1), 0) + row0
    cols = []
    for s in range(nseg):
        lo = cu_ref[s]
        hi = cu_ref[s + 1]
        cols.append(((r >= lo) & (r < hi)).astype(jnp.float32))
    return jnp.concatenate(cols, axis=1)


def _fused_kernel(cu_ref, x_ref, gamma_ref, beta_ref, wt_ref, b_ref,
                  y_ref, xc_ref, sums_ref, sq_ref, scale_ref, bias2_ref,
                  *, rows, nblocks):
    p = pl.program_id(0)
    i = pl.program_id(1)

    @pl.when(p == 0)
    def _stats():
        @pl.when(i == 0)
        def _init():
            sums_ref[...] = jnp.zeros_like(sums_ref)
            sq_ref[...] = jnp.zeros_like(sq_ref)

        x = x_ref[...]
        xc_ref[pl.ds(i * rows, rows), :] = x
        oh = _onehot(cu_ref, i * rows, rows, B)
        ohT = oh.T
        sums_ref[...] += jax.lax.dot(ohT, x, preferred_element_type=jnp.float32)
        sq_ref[...] += jax.lax.dot(ohT, x * x,
                                   preferred_element_type=jnp.float32)

        @pl.when(i == nblocks - 1)
        def _finalize():
            cnts = []
            for s in range(B):
                cnts.append((cu_ref[s + 1] - cu_ref[s]).astype(jnp.float32))
            counts = jnp.stack(cnts).reshape(B, 1)
            mean = sums_ref[...] / counts
            var = sq_ref[...] / counts - mean * mean
            scale = gamma_ref[...] * jax.lax.rsqrt(var + EPS)
            bias = beta_ref[...] - mean * scale
            scale_ref[...] = scale
            bias2_ref[...] = (
                jax.lax.dot(bias, wt_ref[...],
                            preferred_element_type=jnp.float32)
                + b_ref[...])

    @pl.when(p == 1)
    def _apply():
        oh = _onehot(cu_ref, i * rows, rows, B)
        scale_b = jax.lax.dot(oh, scale_ref[...],
                              preferred_element_type=jnp.float32)
        bias_b = jax.lax.dot(oh, bias2_ref[...],
                             preferred_element_type=jnp.float32)
        xs = xc_ref[pl.ds(i * rows, rows), :] * scale_b
        y_ref[...] = jax.lax.dot(xs, wt_ref[...],
                                 preferred_element_type=jnp.float32) + bias_b


@jax.jit
def kernel(flat_features, cu_seqlens, gamma, beta, W, b):
    T, C = flat_features.shape
    WT = W.T
    gamma2 = gamma.reshape(1, C)
    beta2 = beta.reshape(1, C)
    b2 = b.reshape(1, C)

    rows = 1024
    nb = T // rows
    grid_spec = pltpu.PrefetchScalarGridSpec(
        num_scalar_prefetch=1,
        grid=(2, nb),
        in_specs=[
            pl.BlockSpec((rows, C), lambda p, i, cu: (i * (1 - p), 0)),
            pl.BlockSpec((1, C), lambda p, i, cu: (0, 0)),
            pl.BlockSpec((1, C), lambda p, i, cu: (0, 0)),
            pl.BlockSpec((C, C), lambda p, i, cu: (0, 0)),
            pl.BlockSpec((1, C), lambda p, i, cu: (0, 0)),
        ],
        out_specs=pl.BlockSpec((rows, C), lambda p, i, cu: (i * p, 0)),
        scratch_shapes=[
            pltpu.VMEM((T, C), jnp.float32),
            pltpu.VMEM((B, C), jnp.float32),
            pltpu.VMEM((B, C), jnp.float32),
            pltpu.VMEM((B, C), jnp.float32),
            pltpu.VMEM((B, C), jnp.float32),
        ],
    )
    y = pl.pallas_call(
        functools.partial(_fused_kernel, rows=rows, nblocks=nb),
        grid_spec=grid_spec,
        out_shape=jax.ShapeDtypeStruct((T, C), jnp.float32),
    )(cu_seqlens, flat_features, gamma2, beta2, WT, b2)
    return y


# speedup vs baseline: 12.3704x; 1.2801x over previous
import functools

import jax
import jax.numpy as jnp
from jax.experimental import pallas as pl
from jax.experimental.pallas import tpu as pltpu

B = 8
EPS = 1e-5


def _onehot(cu_v, row0, rows):
    r2 = jax.lax.broadcasted_iota(jnp.int32, (rows, B), 0) + row0
    lo = cu_v[0:1, 0:B]
    hi = cu_v[0:1, 1:B + 1]
    return ((r2 >= lo) & (r2 < hi)).astype(jnp.float32)


def _fused_kernel(cu_ref, x_ref, cu_v_ref, gamma_ref, beta_ref, wt_ref, b_ref,
                  y_ref, xc_ref, sums_ref, sq_ref, scale_ref, bias2_ref,
                  *, rows, nblocks):
    p = pl.program_id(0)
    i = pl.program_id(1)

    @pl.when(p == 0)
    def _stats():
        @pl.when(i == 0)
        def _init():
            sums_ref[...] = jnp.zeros_like(sums_ref)
            sq_ref[...] = jnp.zeros_like(sq_ref)

        x = x_ref[...]
        xc_ref[pl.ds(i * rows, rows), :] = x
        oh = _onehot(cu_v_ref[...], i * rows, rows)
        ohT = oh.T
        sums_ref[...] += jax.lax.dot(ohT, x, preferred_element_type=jnp.float32)
        sq_ref[...] += jax.lax.dot(ohT, x * x,
                                   preferred_element_type=jnp.float32)

        @pl.when(i == nblocks - 1)
        def _finalize():
            cnts = []
            for s in range(B):
                cnts.append((cu_ref[s + 1] - cu_ref[s]).astype(jnp.float32))
            counts = jnp.stack(cnts).reshape(B, 1)
            mean = sums_ref[...] / counts
            var = sq_ref[...] / counts - mean * mean
            scale = gamma_ref[...] * jax.lax.rsqrt(var + EPS)
            bias = beta_ref[...] - mean * scale
            scale_ref[...] = scale
            bias2_ref[...] = (
                jax.lax.dot(bias, wt_ref[...],
                            preferred_element_type=jnp.float32)
                + b_ref[...])

    @pl.when(p == 1)
    def _apply():
        oh = _onehot(cu_v_ref[...], i * rows, rows)
        scale_b = jax.lax.dot(oh, scale_ref[...],
                              preferred_element_type=jnp.float32)
        bias_b = jax.lax.dot(oh, bias2_ref[...],
                             preferred_element_type=jnp.float32)
        xs = xc_ref[pl.ds(i * rows, rows), :] * scale_b
        y_ref[...] = jax.lax.dot(xs, wt_ref[...],
                                 preferred_element_type=jnp.float32) + bias_b


@jax.jit
def kernel(flat_features, cu_seqlens, gamma, beta, W, b):
    T, C = flat_features.shape
    WT = W.T
    gamma2 = gamma.reshape(1, C)
    beta2 = beta.reshape(1, C)
    b2 = b.reshape(1, C)
    cu_pad = jnp.zeros((8, 128), jnp.int32).at[0, :B + 1].set(cu_seqlens)

    rows = 2048
    nb = T // rows
    grid_spec = pltpu.PrefetchScalarGridSpec(
        num_scalar_prefetch=1,
        grid=(2, nb),
        in_specs=[
            pl.BlockSpec((rows, C), lambda p, i, cu: (i * (1 - p), 0)),
            pl.BlockSpec((8, 128), lambda p, i, cu: (0, 0)),
            pl.BlockSpec((1, C), lambda p, i, cu: (0, 0)),
            pl.BlockSpec((1, C), lambda p, i, cu: (0, 0)),
            pl.BlockSpec((C, C), lambda p, i, cu: (0, 0)),
            pl.BlockSpec((1, C), lambda p, i, cu: (0, 0)),
        ],
        out_specs=pl.BlockSpec((rows, C), lambda p, i, cu: (i * p, 0)),
        scratch_shapes=[
            pltpu.VMEM((T, C), jnp.float32),
            pltpu.VMEM((B, C), jnp.float32),
            pltpu.VMEM((B, C), jnp.float32),
            pltpu.VMEM((B, C), jnp.float32),
            pltpu.VMEM((B, C), jnp.float32),
        ],
    )
    y = pl.pallas_call(
        functools.partial(_fused_kernel, rows=rows, nblocks=nb),
        grid_spec=grid_spec,
        out_shape=jax.ShapeDtypeStruct((T, C), jnp.float32),
    )(cu_seqlens, flat_features, cu_pad, gamma2, beta2, WT, b2)
    return y


# speedup vs baseline: 12.6112x; 1.0195x over previous
import functools

import jax
import jax.numpy as jnp
from jax.experimental import pallas as pl
from jax.experimental.pallas import tpu as pltpu

B = 8
EPS = 1e-5

_DN_T = (((0,), (0,)), ((), ()))


def _onehot_t(lo_ref, hi_ref, row0, rows):
    r2 = jax.lax.broadcasted_iota(jnp.int32, (B, rows), 1) + row0
    lo = lo_ref[:, 0:1]
    hi = hi_ref[:, 0:1]
    return ((r2 >= lo) & (r2 < hi)).astype(jnp.float32)


def _fused_kernel(cu_ref, x_ref, lo_ref, hi_ref, gamma_ref, beta_ref, wt_ref,
                  b_ref, y_ref, xc_ref, sums_ref, sq_ref, scale_ref,
                  bias2_ref, *, rows, nblocks):
    p = pl.program_id(0)
    i = pl.program_id(1)

    @pl.when(p == 0)
    def _stats():
        @pl.when(i == 0)
        def _init():
            sums_ref[...] = jnp.zeros_like(sums_ref)
            sq_ref[...] = jnp.zeros_like(sq_ref)

        x = x_ref[...]
        xc_ref[pl.ds(i * rows, rows), :] = x
        ohT = _onehot_t(lo_ref, hi_ref, i * rows, rows)
        sums_ref[...] += jax.lax.dot(ohT, x, preferred_element_type=jnp.float32)
        sq_ref[...] += jax.lax.dot(ohT, x * x,
                                   preferred_element_type=jnp.float32)

        @pl.when(i == nblocks - 1)
        def _finalize():
            cnts = []
            for s in range(B):
                cnts.append((cu_ref[s + 1] - cu_ref[s]).astype(jnp.float32))
            counts = jnp.stack(cnts).reshape(B, 1)
            mean = sums_ref[...] / counts
            var = sq_ref[...] / counts - mean * mean
            scale = gamma_ref[...] * jax.lax.rsqrt(var + EPS)
            bias = beta_ref[...] - mean * scale
            scale_ref[...] = scale
            bias2_ref[...] = (
                jax.lax.dot(bias, wt_ref[...],
                            preferred_element_type=jnp.float32)
                + b_ref[...])

    @pl.when(p == 1)
    def _apply():
        ohT = _onehot_t(lo_ref, hi_ref, i * rows, rows)
        scale_b = jax.lax.dot_general(
            ohT, scale_ref[...], _DN_T,
            preferred_element_type=jnp.float32)
        bias_b = jax.lax.dot_general(
            ohT, bias2_ref[...], _DN_T,
            preferred_element_type=jnp.float32)
        xs = xc_ref[pl.ds(i * rows, rows), :] * scale_b
        y_ref[...] = jax.lax.dot(xs, wt_ref[...],
                                 preferred_element_type=jnp.float32) + bias_b


@jax.jit
def kernel(flat_features, cu_seqlens, gamma, beta, W, b):
    T, C = flat_features.shape
    WT = W.T
    gamma2 = gamma.reshape(1, C)
    beta2 = beta.reshape(1, C)
    b2 = b.reshape(1, C)
    lo_b = jnp.broadcast_to(cu_seqlens[:B, None], (B, 128)).astype(jnp.int32)
    hi_b = jnp.broadcast_to(cu_seqlens[1:B + 1, None], (B, 128)).astype(jnp.int32)

    rows = 2048
    nb = T // rows
    grid_spec = pltpu.PrefetchScalarGridSpec(
        num_scalar_prefetch=1,
        grid=(2, nb),
        in_specs=[
            pl.BlockSpec((rows, C),
                         lambda p, i, cu: (i * (1 - p) + (nb - 1) * p, 0)),
            pl.BlockSpec((B, 128), lambda p, i, cu: (0, 0)),
            pl.BlockSpec((B, 128), lambda p, i, cu: (0, 0)),
            pl.BlockSpec((1, C), lambda p, i, cu: (0, 0)),
            pl.BlockSpec((1, C), lambda p, i, cu: (0, 0)),
            pl.BlockSpec((C, C), lambda p, i, cu: (0, 0)),
            pl.BlockSpec((1, C), lambda p, i, cu: (0, 0)),
        ],
        out_specs=pl.BlockSpec((rows, C), lambda p, i, cu: (i * p, 0)),
        scratch_shapes=[
            pltpu.VMEM((T, C), jnp.float32),
            pltpu.VMEM((B, C), jnp.float32),
            pltpu.VMEM((B, C), jnp.float32),
            pltpu.VMEM((B, C), jnp.float32),
            pltpu.VMEM((B, C), jnp.float32),
        ],
    )
    y = pl.pallas_call(
        functools.partial(_fused_kernel, rows=rows, nblocks=nb),
        grid_spec=grid_spec,
        out_shape=jax.ShapeDtypeStruct((T, C), jnp.float32),
    )(cu_seqlens, flat_features, lo_b, hi_b, gamma2, beta2, WT, b2)
    return y


# speedup vs baseline: 12.7275x; 1.0092x over previous
import functools

import jax
import jax.numpy as jnp
from jax.experimental import pallas as pl
from jax.experimental.pallas import tpu as pltpu

B = 8
EPS = 1e-5

_DN_T = (((0,), (0,)), ((), ()))


def _onehot_t(lo_ref, hi_ref, row0, rows, nseg, dtype=jnp.float32):
    r2 = jax.lax.broadcasted_iota(jnp.int32, (nseg, rows), 1) + row0
    lo = lo_ref[:, 0:1]
    hi = hi_ref[:, 0:1]
    return ((r2 >= lo) & (r2 < hi)).astype(dtype)


def _split_bf16(a):
    hi = a.astype(jnp.bfloat16)
    lo = (a - hi.astype(jnp.float32)).astype(jnp.bfloat16)
    return jnp.concatenate([hi, lo], axis=0)


def _fused_kernel(cu_ref, x_ref, lo_ref, hi_ref, gamma_ref, beta_ref, wt_ref,
                  b_ref, y_ref, xc_ref, sums_ref, sq_ref, scale_ref,
                  bias2_ref, *, rows, nblocks):
    p = pl.program_id(0)
    i = pl.program_id(1)

    @pl.when(p == 0)
    def _stats():
        @pl.when(i == 0)
        def _init():
            sums_ref[...] = jnp.zeros_like(sums_ref)
            sq_ref[...] = jnp.zeros_like(sq_ref)

        x = x_ref[...]
        xc_ref[pl.ds(i * rows, rows), :] = x
        ohT = _onehot_t(lo_ref[0:B], hi_ref[0:B], i * rows, rows, B)
        sums_ref[...] += jax.lax.dot(ohT, x, preferred_element_type=jnp.float32)
        sq_ref[...] += jax.lax.dot(ohT, x * x,
                                   preferred_element_type=jnp.float32)

        @pl.when(i == nblocks - 1)
        def _finalize():
            cnts = []
            for s in range(B):
                cnts.append((cu_ref[s + 1] - cu_ref[s]).astype(jnp.float32))
            counts = jnp.stack(cnts).reshape(B, 1)
            mean = sums_ref[...] / counts
            var = sq_ref[...] / counts - mean * mean
            scale = gamma_ref[...] * jax.lax.rsqrt(var + EPS)
            bias = beta_ref[...] - mean * scale
            bias2 = (jax.lax.dot(bias, wt_ref[...],
                                 preferred_element_type=jnp.float32)
                     + b_ref[...])
            scale_ref[...] = _split_bf16(scale)
            bias2_ref[...] = _split_bf16(bias2)

    @pl.when(p == 1)
    def _apply():
        oh2 = _onehot_t(lo_ref, hi_ref, i * rows, rows, 2 * B,
                        jnp.bfloat16)
        scale_b = jax.lax.dot_general(
            oh2, scale_ref[...], _DN_T,
            preferred_element_type=jnp.float32)
        bias_b = jax.lax.dot_general(
            oh2, bias2_ref[...], _DN_T,
            preferred_element_type=jnp.float32)
        xs = xc_ref[pl.ds(i * rows, rows), :] * scale_b
        y_ref[...] = jax.lax.dot(xs, wt_ref[...],
                                 preferred_element_type=jnp.float32) + bias_b


@jax.jit
def kernel(flat_features, cu_seqlens, gamma, beta, W, b):
    T, C = flat_features.shape
    WT = W.T
    gamma2 = gamma.reshape(1, C)
    beta2 = beta.reshape(1, C)
    b2 = b.reshape(1, C)
    lo1 = jnp.broadcast_to(cu_seqlens[:B, None], (B, 128)).astype(jnp.int32)
    hi1 = jnp.broadcast_to(cu_seqlens[1:B + 1, None], (B, 128)).astype(jnp.int32)
    lo_b = jnp.concatenate([lo1, lo1], axis=0)
    hi_b = jnp.concatenate([hi1, hi1], axis=0)

    rows = 2048
    nb = T // rows
    grid_spec = pltpu.PrefetchScalarGridSpec(
        num_scalar_prefetch=1,
        grid=(2, nb),
        in_specs=[
            pl.BlockSpec((rows, C),
                         lambda p, i, cu: (i * (1 - p) + (nb - 1) * p, 0)),
            pl.BlockSpec((2 * B, 128), lambda p, i, cu: (0, 0)),
            pl.BlockSpec((2 * B, 128), lambda p, i, cu: (0, 0)),
            pl.BlockSpec((1, C), lambda p, i, cu: (0, 0)),
            pl.BlockSpec((1, C), lambda p, i, cu: (0, 0)),
            pl.BlockSpec((C, C), lambda p, i, cu: (0, 0)),
            pl.BlockSpec((1, C), lambda p, i, cu: (0, 0)),
        ],
        out_specs=pl.BlockSpec((rows, C), lambda p, i, cu: (i * p, 0)),
        scratch_shapes=[
            pltpu.VMEM((T, C), jnp.float32),
            pltpu.VMEM((B, C), jnp.float32),
            pltpu.VMEM((B, C), jnp.float32),
            pltpu.VMEM((2 * B, C), jnp.bfloat16),
            pltpu.VMEM((2 * B, C), jnp.bfloat16),
        ],
    )
    y = pl.pallas_call(
        functools.partial(_fused_kernel, rows=rows, nblocks=nb),
        grid_spec=grid_spec,
        out_shape=jax.ShapeDtypeStruct((T, C), jnp.float32),
    )(cu_seqlens, flat_features, lo_b, hi_b, gamma2, beta2, WT, b2)
    return y
